# Initial kernel scaffold; baseline (speedup 1.0000x reference)
import jax
import jax.numpy as jnp
from jax.experimental import pallas as pl


def kernel(pred, target):
    raise NotImplementedError("write your pallas kernel here")


# speedup vs baseline: 16.9684x; 16.9684x over previous
import functools

import jax
import jax.numpy as jnp
from jax import lax
from jax.experimental import pallas as pl
from jax.experimental.pallas import tpu as pltpu
from jax.experimental.pallas import tpu_sc as plsc

N = 8 * 512 * 512
NW = 32
PER_W = N // NW
CS = 4096
NCHUNK = PER_W // CS
NBBITS = 15
NB = 1 << NBBITS
SHIFT = 32 - NBBITS
HALF = NB // 2
L = 16

_mesh = plsc.VectorSubcoreMesh(core_axis_name="c", subcore_axis_name="s")


def _bucket_idx(p, t):
    tf = t.astype(jnp.float32)
    e = 1.0 - p * (2.0 * tf - 1.0)
    s = lax.bitcast_convert_type(e, jnp.int32)
    key = s ^ ((s >> 31) & jnp.int32(0x7FFFFFFF))
    b = (key >> SHIFT) + HALF
    return b + t * NB, e


@functools.partial(
    pl.kernel,
    out_type=jax.ShapeDtypeStruct((NW, 2 * NB), jnp.int32),
    mesh=_mesh,
    compiler_params=pltpu.CompilerParams(needs_layout_passes=False),
    scratch_types=[
        pltpu.VMEM((2 * NB,), jnp.int32),
        pltpu.VMEM((CS,), jnp.float32),
        pltpu.VMEM((CS,), jnp.int32),
    ],
)
def _hist_kernel(pred_hbm, tgt_hbm, out_hbm, hist, pbuf, tbuf):
    wid = lax.axis_index("s") * 2 + lax.axis_index("c")
    base = wid * PER_W

    def zero_body(i, _):
        hist[pl.ds(i * L, L)] = jnp.zeros((L,), jnp.int32)
        return 0

    lax.fori_loop(0, 2 * NB // L, zero_body, 0)

    ones = jnp.ones((L,), jnp.int32)

    def chunk(ci):
        pltpu.sync_copy(pred_hbm.at[pl.ds(base + ci * CS, CS)], pbuf)
        pltpu.sync_copy(tgt_hbm.at[pl.ds(base + ci * CS, CS)], tbuf)

        def body(j, _):
            p = pbuf[pl.ds(j * L, L)]
            t = tbuf[pl.ds(j * L, L)]
            idx, _e = _bucket_idx(p, t)
            plsc.addupdate_scatter(hist, [idx], ones)
            return 0

        lax.fori_loop(0, CS // L, body, 0)

    for ci in range(NCHUNK):
        chunk(ci)

    pltpu.sync_copy(hist, out_hbm.at[wid])


def _weights_body(hist_ref, w_ref):
    h = jnp.sum(hist_ref[...], axis=0)
    lane = lax.broadcasted_iota(jnp.int32, (512, 128), 1)
    c = h
    k = 1
    while k < 128:
        c = c + jnp.where(lane >= k, pltpu.roll(c, k, 1), 0)
        k *= 2
    row_tot = c[:, 127:128]
    row = lax.broadcasted_iota(jnp.int32, (512, 1), 0)
    rmod = row & 255
    r = row_tot
    k = 1
    while k < 256:
        r = r + jnp.where(rmod >= k, pltpu.roll(r, k, 0), 0)
        k *= 2
    excl = r - row_tot
    cincl = c + excl
    c0 = cincl[0:256, :]
    c1 = cincl[256:512, :]
    h0 = h[0:256, :]
    h1 = h[256:512, :]
    ftot = c0[255, 127].astype(jnp.float32)
    ptot = c1[255, 127].astype(jnp.float32)
    c0f = c0.astype(jnp.float32)
    d1 = ptot + ftot - c0f
    d2 = d1 + h0.astype(jnp.float32)
    q = (c1 - h1).astype(jnp.float32)
    d1s = jnp.maximum(d1, 1.0)
    w1 = jnp.where(d1 > 0, 1.0 / d1s, 0.0)
    w0f = jnp.where(h0 > 0, 1.0 / jnp.maximum(h0.astype(jnp.float32), 1.0), 0.0)
    w0 = jnp.where(d1 > 0, q / (d1s * jnp.maximum(d2, 1.0)), w0f)
    w_ref[0] = w0
    w_ref[1] = w1


_weights_kernel = pl.pallas_call(
    _weights_body,
    out_shape=jax.ShapeDtypeStruct((2, 256, 128), jnp.float32),
)


@functools.partial(
    pl.kernel,
    out_type=jax.ShapeDtypeStruct((NW, L), jnp.float32),
    mesh=_mesh,
    compiler_params=pltpu.CompilerParams(needs_layout_passes=False),
    scratch_types=[
        pltpu.VMEM((2 * NB,), jnp.float32),
        pltpu.VMEM((CS,), jnp.float32),
        pltpu.VMEM((CS,), jnp.int32),
        pltpu.VMEM((L,), jnp.float32),
    ],
)
def _loss_kernel(pred_hbm, tgt_hbm, w_hbm, out_hbm, wtab, pbuf, tbuf, accbuf):
    wid = lax.axis_index("s") * 2 + lax.axis_index("c")
    base = wid * PER_W
    pltpu.sync_copy(w_hbm, wtab)

    def chunk(ci, acc):
        pltpu.sync_copy(pred_hbm.at[pl.ds(base + ci * CS, CS)], pbuf)
        pltpu.sync_copy(tgt_hbm.at[pl.ds(base + ci * CS, CS)], tbuf)

        def body(j, a):
            p = pbuf[pl.ds(j * L, L)]
            t = tbuf[pl.ds(j * L, L)]
            idx, e = _bucket_idx(p, t)
            w = plsc.load_gather(wtab, [idx])
            return a + jnp.maximum(e, 0.0) * w

        return lax.fori_loop(0, CS // L, body, acc)

    acc = jnp.zeros((L,), jnp.float32)
    for ci in range(NCHUNK):
        acc = chunk(ci, acc)
    accbuf[...] = acc
    pltpu.sync_copy(accbuf, out_hbm.at[wid])


def _sum_body(x_ref, o_ref):
    o_ref[...] = jnp.sum(x_ref[...], keepdims=True)


_sum_kernel = pl.pallas_call(
    _sum_body,
    out_shape=jax.ShapeDtypeStruct((1, 1), jnp.float32),
)


def kernel(pred, target):
    p = pred.reshape(N)
    t = target.reshape(N).astype(jnp.int32)
    hists = _hist_kernel(p, t)
    w = _weights_kernel(hists.reshape(NW, 512, 128))
    partials = _loss_kernel(p, t, w.reshape(2 * NB))
    return _sum_kernel(partials).reshape(())


# speedup vs baseline: 35.3186x; 2.0814x over previous
import functools

import jax
import jax.numpy as jnp
from jax import lax
from jax.experimental import pallas as pl
from jax.experimental.pallas import tpu as pltpu
from jax.experimental.pallas import tpu_sc as plsc

N = 8 * 512 * 512
NW = 32
PER_W = N // NW
CS = 8192
NCHUNK = PER_W // CS
NBBITS = 15
NB = 1 << NBBITS
SHIFT = 32 - NBBITS
HALF = NB // 2
L = 16

_mesh = plsc.VectorSubcoreMesh(core_axis_name="c", subcore_axis_name="s")


def _bucket_idx(p, t):
    pb = lax.bitcast_convert_type(p, jnp.int32)
    e = 1.0 + lax.bitcast_convert_type(pb ^ (t << 31), jnp.float32)
    s = lax.bitcast_convert_type(e, jnp.int32)
    key = s ^ ((s >> 31) & jnp.int32(0x7FFFFFFF))
    b = (key >> SHIFT) + HALF
    return b + (t << NBBITS), e


@functools.partial(
    pl.kernel,
    out_type=jax.ShapeDtypeStruct((NW, 2 * NB), jnp.int32),
    mesh=_mesh,
    compiler_params=pltpu.CompilerParams(needs_layout_passes=False),
    scratch_types=[
        pltpu.VMEM((2 * NB,), jnp.int32),
        pltpu.VMEM((2, CS), jnp.float32),
        pltpu.VMEM((2, CS), jnp.int32),
        pltpu.SemaphoreType.DMA,
        pltpu.SemaphoreType.DMA,
    ],
)
def _hist_kernel(pred_hbm, tgt_hbm, out_hbm, hist, pbuf, tbuf, sem0, sem1):
    wid = lax.axis_index("s") * 2 + lax.axis_index("c")
    base = wid * PER_W
    sems = (sem0, sem1)

    def start(ci):
        sl = pl.ds(base + ci * CS, CS)
        k = ci % 2
        dp = pltpu.async_copy(pred_hbm.at[sl], pbuf.at[k], sems[k])
        dt = pltpu.async_copy(tgt_hbm.at[sl], tbuf.at[k], sems[k])
        return dp, dt

    pend = start(0)

    @plsc.parallel_loop(0, 2 * NB, L, unroll=8)
    def _zero(i):
        hist[pl.ds(i, L)] = jnp.zeros((L,), jnp.int32)

    ones = jnp.ones((L,), jnp.int32)

    for ci in range(NCHUNK):
        k = ci % 2
        nxt = start(ci + 1) if ci + 1 < NCHUNK else None
        pend[0].wait()
        pend[1].wait()
        pend = nxt

        @plsc.parallel_loop(0, CS, L, unroll=4)
        def _body(j):
            p = pbuf[k, pl.ds(j, L)]
            t = tbuf[k, pl.ds(j, L)]
            idx, _e = _bucket_idx(p, t)
            plsc.addupdate_scatter(hist, [idx], ones)

    pltpu.sync_copy(hist, out_hbm.at[wid])


def _weights_body(hist_ref, w_ref):
    h = jnp.sum(hist_ref[...], axis=0)
    lane = lax.broadcasted_iota(jnp.int32, (512, 128), 1)
    c = h
    k = 1
    while k < 128:
        c = c + jnp.where(lane >= k, pltpu.roll(c, k, 1), 0)
        k *= 2
    row_tot = c[:, 127:128]
    row = lax.broadcasted_iota(jnp.int32, (512, 1), 0)
    rmod = row & 255
    r = row_tot
    k = 1
    while k < 256:
        r = r + jnp.where(rmod >= k, pltpu.roll(r, k, 0), 0)
        k *= 2
    excl = r - row_tot
    cincl = c + excl
    c0 = cincl[0:256, :]
    c1 = cincl[256:512, :]
    h0 = h[0:256, :]
    h1 = h[256:512, :]
    ftot = c0[255, 127].astype(jnp.float32)
    ptot = c1[255, 127].astype(jnp.float32)
    c0f = c0.astype(jnp.float32)
    d1 = ptot + ftot - c0f
    d2 = d1 + h0.astype(jnp.float32)
    q = (c1 - h1).astype(jnp.float32)
    d1s = jnp.maximum(d1, 1.0)
    w1 = jnp.where(d1 > 0, 1.0 / d1s, 0.0)
    w0f = jnp.where(h0 > 0, 1.0 / jnp.maximum(h0.astype(jnp.float32), 1.0), 0.0)
    w0 = jnp.where(d1 > 0, q / (d1s * jnp.maximum(d2, 1.0)), w0f)
    w_ref[0] = w0
    w_ref[1] = w1


_weights_kernel = pl.pallas_call(
    _weights_body,
    out_shape=jax.ShapeDtypeStruct((2, 256, 128), jnp.float32),
)


@functools.partial(
    pl.kernel,
    out_type=jax.ShapeDtypeStruct((NW, L), jnp.float32),
    mesh=_mesh,
    compiler_params=pltpu.CompilerParams(needs_layout_passes=False),
    scratch_types=[
        pltpu.VMEM((2 * NB,), jnp.float32),
        pltpu.VMEM((2, CS), jnp.float32),
        pltpu.VMEM((2, CS), jnp.int32),
        pltpu.VMEM((L,), jnp.float32),
        pltpu.SemaphoreType.DMA,
        pltpu.SemaphoreType.DMA,
        pltpu.SemaphoreType.DMA,
    ],
)
def _loss_kernel(pred_hbm, tgt_hbm, w_hbm, out_hbm, wtab, pbuf, tbuf, accbuf,
                 sem0, sem1, semw):
    wid = lax.axis_index("s") * 2 + lax.axis_index("c")
    base = wid * PER_W
    sems = (sem0, sem1)
    dw = pltpu.async_copy(w_hbm, wtab, semw)

    def start(ci):
        sl = pl.ds(base + ci * CS, CS)
        k = ci % 2
        dp = pltpu.async_copy(pred_hbm.at[sl], pbuf.at[k], sems[k])
        dt = pltpu.async_copy(tgt_hbm.at[sl], tbuf.at[k], sems[k])
        return dp, dt

    pend = start(0)
    dw.wait()

    acc = jnp.zeros((L,), jnp.float32)
    for ci in range(NCHUNK):
        k = ci % 2
        nxt = start(ci + 1) if ci + 1 < NCHUNK else None
        pend[0].wait()
        pend[1].wait()
        pend = nxt

        @plsc.parallel_loop(0, CS, L, unroll=4, carry=acc)
        def _body(j, a):
            p = pbuf[k, pl.ds(j, L)]
            t = tbuf[k, pl.ds(j, L)]
            idx, e = _bucket_idx(p, t)
            w = plsc.load_gather(wtab, [idx])
            return a + jnp.maximum(e, 0.0) * w

        acc = _body
    accbuf[...] = acc
    pltpu.sync_copy(accbuf, out_hbm.at[wid])


def _sum_body(x_ref, o_ref):
    o_ref[...] = jnp.sum(x_ref[...], keepdims=True)


_sum_kernel = pl.pallas_call(
    _sum_body,
    out_shape=jax.ShapeDtypeStruct((1, 1), jnp.float32),
)


def kernel(pred, target):
    p = pred.reshape(N)
    t = target.reshape(N).astype(jnp.int32)
    hists = _hist_kernel(p, t)
    w = _weights_kernel(hists.reshape(NW, 512, 128))
    partials = _loss_kernel(p, t, w.reshape(2 * NB))
    return _sum_kernel(partials).reshape(())


# speedup vs baseline: 49.8384x; 1.4111x over previous
import functools

import jax
import jax.numpy as jnp
from jax import lax
from jax.experimental import pallas as pl
from jax.experimental.pallas import tpu as pltpu
from jax.experimental.pallas import tpu_sc as plsc

N = 8 * 512 * 512
NW = 32
PER_W = N // NW
CS = 8192
NCHUNK = PER_W // CS
NBBITS = 14
NB = 1 << NBBITS
SHIFT = 32 - NBBITS
HALF = NB // 2
L = 16
ROWS = 2 * NB // 128

_mesh = plsc.VectorSubcoreMesh(core_axis_name="c", subcore_axis_name="s")


@functools.partial(
    pl.kernel,
    out_type=jax.ShapeDtypeStruct((2, 2, ROWS, 128), jnp.float32),
    mesh=_mesh,
    compiler_params=pltpu.CompilerParams(needs_layout_passes=False),
    scratch_types=[
        pltpu.VMEM((ROWS, 128), jnp.float32),
        pltpu.VMEM((ROWS, 128), jnp.float32),
        pltpu.VMEM((2, CS), jnp.float32),
        pltpu.VMEM((2, CS), jnp.int32),
        pltpu.VMEM((2, 128), jnp.int32),
        pltpu.VMEM_SHARED((ROWS, 128), jnp.float32),
        pltpu.VMEM_SHARED((ROWS, 128), jnp.float32),
        pltpu.SemaphoreType.DMA,
        pltpu.SemaphoreType.DMA,
    ],
)
def _hist_kernel(pred_hbm, tgt_hbm, out_hbm, cnt, sm, pbuf, tbuf, idxrows,
                 sh_cnt, sh_sm, sem0, sem1):
    core = lax.axis_index("c")
    sid = lax.axis_index("s")
    wid = sid * 2 + core
    base = wid * PER_W
    sems = (sem0, sem1)

    def start(ci):
        sl = pl.ds(base + ci * CS, CS)
        k = ci % 2
        dp = pltpu.async_copy(pred_hbm.at[sl], pbuf.at[k], sems[k])
        dt = pltpu.async_copy(tgt_hbm.at[sl], tbuf.at[k], sems[k])
        return dp, dt

    pend = start(0)

    zeros = jnp.zeros((L,), jnp.float32)

    @plsc.parallel_loop(0, ROWS * 128, L, unroll=8)
    def _zero(i):
        cnt[i >> 7, pl.ds(i & 127, L)] = zeros
        sm[i >> 7, pl.ds(i & 127, L)] = zeros

    iot = lax.iota(jnp.int32, L)
    for k in range(2):
        for j in range(128 // L):
            idxrows[k, pl.ds(j * L, L)] = iot + (k * 128 + j * L)

    ones = jnp.ones((L,), jnp.float32)

    for ci in range(NCHUNK):
        k = ci % 2
        nxt = start(ci + 1) if ci + 1 < NCHUNK else None
        pend[0].wait()
        pend[1].wait()
        pend = nxt

        @plsc.parallel_loop(0, CS, L, unroll=4)
        def _body(j):
            p = pbuf[k, pl.ds(j, L)]
            t = tbuf[k, pl.ds(j, L)]
            pb = lax.bitcast_convert_type(p, jnp.int32)
            e = 1.0 + lax.bitcast_convert_type(pb ^ (t << 31), jnp.float32)
            s = lax.bitcast_convert_type(e, jnp.int32)
            key = s ^ ((s >> 31) & jnp.int32(0x7FFFFFFF))
            idx = (key >> SHIFT) + HALF + (t << NBBITS)
            row = idx >> 7
            lane = idx & 127
            plsc.addupdate_scatter(cnt, [row, lane], ones)
            plsc.addupdate_scatter(sm, [row, lane], jnp.maximum(e, 0.0))

    plsc.subcore_barrier()

    @pl.when(sid == 0)
    def _seed():
        pltpu.sync_copy(cnt, sh_cnt)
        pltpu.sync_copy(sm, sh_sm)

    plsc.subcore_barrier()

    @pl.when(sid != 0)
    def _accum():
        for k in range(2):
            rows = pl.ds(k * 128, 128)
            pltpu.sync_copy(cnt.at[rows], sh_cnt.at[idxrows.at[k]], add=True)
            pltpu.sync_copy(sm.at[rows], sh_sm.at[idxrows.at[k]], add=True)

    plsc.subcore_barrier()

    @pl.when(sid == 0)
    def _flush():
        pltpu.sync_copy(sh_cnt, out_hbm.at[core, 0])
        pltpu.sync_copy(sh_sm, out_hbm.at[core, 1])


def _final_body(hin_ref, o_ref):
    h = hin_ref[0] + hin_ref[1]
    cnt = h[0]
    sm = h[1]
    lane = lax.broadcasted_iota(jnp.int32, (ROWS, 128), 1)
    c = cnt
    k = 1
    while k < 128:
        c = c + jnp.where(lane >= k, pltpu.roll(c, k, 1), 0.0)
        k *= 2
    row_tot = c[:, 127:128]
    row = lax.broadcasted_iota(jnp.int32, (ROWS, 1), 0)
    rmod = row & 127
    r = row_tot
    k = 1
    while k < 128:
        r = r + jnp.where(rmod >= k, pltpu.roll(r, k, 0), 0.0)
        k *= 2
    cincl = c + (r - row_tot)
    c0 = cincl[0:128, :]
    c1 = cincl[128:256, :]
    h0 = cnt[0:128, :]
    h1 = cnt[128:256, :]
    d1 = jnp.float32(N) - c0
    d2 = d1 + h0
    q = c1 - h1
    d1s = jnp.maximum(d1, 1.0)
    w1 = jnp.where(d1 > 0, 1.0 / d1s, 0.0)
    w0f = jnp.where(h0 > 0, 1.0 / jnp.maximum(h0, 1.0), 0.0)
    w0 = jnp.where(d1 > 0, q / (d1s * jnp.maximum(d2, 1.0)), w0f)
    loss = jnp.sum(sm[128:256, :] * w1) + jnp.sum(sm[0:128, :] * w0)
    o_ref[...] = jnp.reshape(loss, (1, 1))


_final_kernel = pl.pallas_call(
    _final_body,
    out_shape=jax.ShapeDtypeStruct((1, 1), jnp.float32),
)


def kernel(pred, target):
    p = pred.reshape(N)
    t = target.reshape(N).astype(jnp.int32)
    hists = _hist_kernel(p, t)
    return _final_kernel(hists).reshape(())


# speedup vs baseline: 53.1591x; 1.0666x over previous
import functools

import jax
import jax.numpy as jnp
from jax import lax
from jax.experimental import pallas as pl
from jax.experimental.pallas import tpu as pltpu
from jax.experimental.pallas import tpu_sc as plsc

N = 8 * 512 * 512
NW = 32
PER_W = N // NW
CS = 8192
NCHUNK = PER_W // CS
NBBITS = 12
NB = 1 << NBBITS
SHIFT = 32 - NBBITS
HALF = NB // 2
L = 16
ROWS = 2 * NB // 128
CLS_ROWS = NB // 128
RT = (ROWS + 127) // 128
RPT = ROWS // RT

_mesh = plsc.VectorSubcoreMesh(core_axis_name="c", subcore_axis_name="s")


@functools.partial(
    pl.kernel,
    out_type=jax.ShapeDtypeStruct((2, 2, ROWS, 128), jnp.float32),
    mesh=_mesh,
    compiler_params=pltpu.CompilerParams(needs_layout_passes=False),
    scratch_types=[
        pltpu.VMEM((ROWS, 128), jnp.float32),
        pltpu.VMEM((ROWS, 128), jnp.float32),
        pltpu.VMEM((2, CS), jnp.float32),
        pltpu.VMEM((2, CS), jnp.int32),
        pltpu.VMEM((RT, RPT), jnp.int32),
        pltpu.VMEM_SHARED((ROWS, 128), jnp.float32),
        pltpu.VMEM_SHARED((ROWS, 128), jnp.float32),
        pltpu.SemaphoreType.DMA,
        pltpu.SemaphoreType.DMA,
    ],
)
def _hist_kernel(pred_hbm, tgt_hbm, out_hbm, cnt, sm, pbuf, tbuf, idxrows,
                 sh_cnt, sh_sm, sem0, sem1):
    core = lax.axis_index("c")
    sid = lax.axis_index("s")
    wid = sid * 2 + core
    base = wid * PER_W
    sems = (sem0, sem1)

    def start(ci):
        sl = pl.ds(base + ci * CS, CS)
        k = ci % 2
        dp = pltpu.async_copy(pred_hbm.at[sl], pbuf.at[k], sems[k])
        dt = pltpu.async_copy(tgt_hbm.at[sl], tbuf.at[k], sems[k])
        return dp, dt

    pend = start(0)

    zeros = jnp.zeros((L,), jnp.float32)

    @plsc.parallel_loop(0, ROWS * 128, L, unroll=8)
    def _zero(i):
        cnt[i >> 7, pl.ds(i & 127, L)] = zeros
        sm[i >> 7, pl.ds(i & 127, L)] = zeros

    iot = lax.iota(jnp.int32, L)
    for k in range(RT):
        for j in range(RPT // L):
            idxrows[k, pl.ds(j * L, L)] = iot + (k * RPT + j * L)

    ones = jnp.ones((L,), jnp.float32)

    for ci in range(NCHUNK):
        k = ci % 2
        nxt = start(ci + 1) if ci + 1 < NCHUNK else None
        pend[0].wait()
        pend[1].wait()
        pend = nxt

        @plsc.parallel_loop(0, CS, L, unroll=4)
        def _body(j):
            p = pbuf[k, pl.ds(j, L)]
            t = tbuf[k, pl.ds(j, L)]
            pb = lax.bitcast_convert_type(p, jnp.int32)
            e = 1.0 + lax.bitcast_convert_type(pb ^ (t << 31), jnp.float32)
            s = lax.bitcast_convert_type(e, jnp.int32)
            key = s ^ ((s >> 31) & jnp.int32(0x7FFFFFFF))
            idx = (key >> SHIFT) + HALF + (t << NBBITS)
            row = idx >> 7
            lane = idx & 127
            plsc.addupdate_scatter(cnt, [row, lane], ones)
            plsc.addupdate_scatter(sm, [row, lane], jnp.maximum(e, 0.0))

    plsc.subcore_barrier()

    @pl.when(sid == 0)
    def _seed():
        pltpu.sync_copy(cnt, sh_cnt)
        pltpu.sync_copy(sm, sh_sm)

    plsc.subcore_barrier()

    @pl.when(sid != 0)
    def _accum():
        for k in range(RT):
            rows = pl.ds(k * RPT, RPT)
            pltpu.sync_copy(cnt.at[rows], sh_cnt.at[idxrows.at[k]], add=True)
            pltpu.sync_copy(sm.at[rows], sh_sm.at[idxrows.at[k]], add=True)

    plsc.subcore_barrier()

    @pl.when(sid == 0)
    def _flush():
        pltpu.sync_copy(sh_cnt, out_hbm.at[core, 0])
        pltpu.sync_copy(sh_sm, out_hbm.at[core, 1])


def _final_body(hin_ref, o_ref):
    h = hin_ref[0] + hin_ref[1]
    cnt = h[0]
    sm = h[1]
    lane = lax.broadcasted_iota(jnp.int32, (ROWS, 128), 1)
    c = cnt
    k = 1
    while k < 128:
        c = c + jnp.where(lane >= k, pltpu.roll(c, k, 1), 0.0)
        k *= 2
    row_tot = c[:, 127:128]
    row = lax.broadcasted_iota(jnp.int32, (ROWS, 1), 0)
    rmod = row & (CLS_ROWS - 1)
    r = row_tot
    k = 1
    while k < CLS_ROWS:
        r = r + jnp.where(rmod >= k, pltpu.roll(r, k, 0), 0.0)
        k *= 2
    cincl = c + (r - row_tot)
    c0 = cincl[0:CLS_ROWS, :]
    c1 = cincl[CLS_ROWS:ROWS, :]
    h0 = cnt[0:CLS_ROWS, :]
    h1 = cnt[CLS_ROWS:ROWS, :]
    d1 = jnp.float32(N) - c0
    d2 = d1 + h0
    q = c1 - h1
    d1s = jnp.maximum(d1, 1.0)
    w1 = jnp.where(d1 > 0, 1.0 / d1s, 0.0)
    w0f = jnp.where(h0 > 0, 1.0 / jnp.maximum(h0, 1.0), 0.0)
    w0 = jnp.where(d1 > 0, q / (d1s * jnp.maximum(d2, 1.0)), w0f)
    loss = jnp.sum(sm[CLS_ROWS:ROWS, :] * w1) + jnp.sum(sm[0:CLS_ROWS, :] * w0)
    o_ref[...] = jnp.reshape(loss, (1, 1))


_final_kernel = pl.pallas_call(
    _final_body,
    out_shape=jax.ShapeDtypeStruct((1, 1), jnp.float32),
)


def kernel(pred, target):
    p = pred.reshape(N)
    t = target.reshape(N).astype(jnp.int32)
    hists = _hist_kernel(p, t)
    return _final_kernel(hists).reshape(())


# speedup vs baseline: 63.0145x; 1.1854x over previous
import functools

import jax
import jax.numpy as jnp
from jax import lax
from jax.experimental import pallas as pl
from jax.experimental.pallas import tpu as pltpu
from jax.experimental.pallas import tpu_sc as plsc

N = 8 * 512 * 512
NW = 32
PER_W = N // NW
CS = 8192
NCHUNK = PER_W // CS
DROWS = N // 128
CROWS = CS // 128
WROWS = PER_W // 128
NBBITS = 12
NB = 1 << NBBITS
SHIFT = 32 - NBBITS
HALF = NB // 2
L = 16
ROWS = 2 * NB // 128
CLS_ROWS = NB // 128
RT = (ROWS + 127) // 128
RPT = ROWS // RT

_mesh = plsc.VectorSubcoreMesh(core_axis_name="c", subcore_axis_name="s")


@functools.partial(
    pl.kernel,
    out_type=jax.ShapeDtypeStruct((2, 2, ROWS, 128), jnp.float32),
    mesh=_mesh,
    compiler_params=pltpu.CompilerParams(needs_layout_passes=False),
    scratch_types=[
        pltpu.VMEM((ROWS, 128), jnp.float32),
        pltpu.VMEM((ROWS, 128), jnp.float32),
        pltpu.VMEM((2, CROWS, 128), jnp.float32),
        pltpu.VMEM((2, CROWS, 128), jnp.int32),
        pltpu.VMEM((RT, RPT), jnp.int32),
        pltpu.VMEM_SHARED((ROWS, 128), jnp.float32),
        pltpu.VMEM_SHARED((ROWS, 128), jnp.float32),
        pltpu.SemaphoreType.DMA,
        pltpu.SemaphoreType.DMA,
    ],
)
def _hist_kernel(pred_hbm, tgt_hbm, out_hbm, cnt, sm, pbuf, tbuf, idxrows,
                 sh_cnt, sh_sm, sem0, sem1):
    core = lax.axis_index("c")
    sid = lax.axis_index("s")
    wid = sid * 2 + core
    base = wid * WROWS
    sems = (sem0, sem1)

    def start(ci):
        sl = pl.ds(base + ci * CROWS, CROWS)
        k = ci % 2
        dp = pltpu.async_copy(pred_hbm.at[sl], pbuf.at[k], sems[k])
        dt = pltpu.async_copy(tgt_hbm.at[sl], tbuf.at[k], sems[k])
        return dp, dt

    pend = start(0)

    zeros = jnp.zeros((L,), jnp.float32)

    @plsc.parallel_loop(0, ROWS * 128, L, unroll=8)
    def _zero(i):
        cnt[i >> 7, pl.ds(i & 127, L)] = zeros
        sm[i >> 7, pl.ds(i & 127, L)] = zeros

    iot = lax.iota(jnp.int32, L)
    for k in range(RT):
        for j in range(RPT // L):
            idxrows[k, pl.ds(j * L, L)] = iot + (k * RPT + j * L)

    ones = jnp.ones((L,), jnp.float32)

    for ci in range(NCHUNK):
        k = ci % 2
        nxt = start(ci + 1) if ci + 1 < NCHUNK else None
        pend[0].wait()
        pend[1].wait()
        pend = nxt

        @plsc.parallel_loop(0, CS, L, unroll=4)
        def _body(j):
            p = pbuf[k, j >> 7, pl.ds(j & 127, L)]
            t = tbuf[k, j >> 7, pl.ds(j & 127, L)]
            pb = lax.bitcast_convert_type(p, jnp.int32)
            e = 1.0 + lax.bitcast_convert_type(pb ^ (t << 31), jnp.float32)
            s = lax.bitcast_convert_type(e, jnp.int32)
            key = s ^ ((s >> 31) & jnp.int32(0x7FFFFFFF))
            idx = (key >> SHIFT) + HALF + (t << NBBITS)
            row = idx >> 7
            lane = idx & 127
            plsc.addupdate_scatter(cnt, [row, lane], ones)
            plsc.addupdate_scatter(sm, [row, lane], jnp.maximum(e, 0.0))

    plsc.subcore_barrier()

    @pl.when(sid == 0)
    def _seed():
        pltpu.sync_copy(cnt, sh_cnt)
        pltpu.sync_copy(sm, sh_sm)

    plsc.subcore_barrier()

    @pl.when(sid != 0)
    def _accum():
        for k in range(RT):
            rows = pl.ds(k * RPT, RPT)
            pltpu.sync_copy(cnt.at[rows], sh_cnt.at[idxrows.at[k]], add=True)
            pltpu.sync_copy(sm.at[rows], sh_sm.at[idxrows.at[k]], add=True)

    plsc.subcore_barrier()

    @pl.when(sid == 0)
    def _flush():
        pltpu.sync_copy(sh_cnt, out_hbm.at[core, 0])
        pltpu.sync_copy(sh_sm, out_hbm.at[core, 1])


def _final_body(hin_ref, o_ref):
    h = hin_ref[0] + hin_ref[1]
    cnt = h[0]
    sm = h[1]
    lane = lax.broadcasted_iota(jnp.int32, (ROWS, 128), 1)
    c = cnt
    k = 1
    while k < 128:
        c = c + jnp.where(lane >= k, pltpu.roll(c, k, 1), 0.0)
        k *= 2
    row_tot = c[:, 127:128]
    row = lax.broadcasted_iota(jnp.int32, (ROWS, 1), 0)
    rmod = row & (CLS_ROWS - 1)
    r = row_tot
    k = 1
    while k < CLS_ROWS:
        r = r + jnp.where(rmod >= k, pltpu.roll(r, k, 0), 0.0)
        k *= 2
    cincl = c + (r - row_tot)
    c0 = cincl[0:CLS_ROWS, :]
    c1 = cincl[CLS_ROWS:ROWS, :]
    h0 = cnt[0:CLS_ROWS, :]
    h1 = cnt[CLS_ROWS:ROWS, :]
    d1 = jnp.float32(N) - c0
    d2 = d1 + h0
    q = c1 - h1
    d1s = jnp.maximum(d1, 1.0)
    w1 = jnp.where(d1 > 0, 1.0 / d1s, 0.0)
    w0f = jnp.where(h0 > 0, 1.0 / jnp.maximum(h0, 1.0), 0.0)
    w0 = jnp.where(d1 > 0, q / (d1s * jnp.maximum(d2, 1.0)), w0f)
    loss = jnp.sum(sm[CLS_ROWS:ROWS, :] * w1) + jnp.sum(sm[0:CLS_ROWS, :] * w0)
    o_ref[...] = jnp.reshape(loss, (1, 1))


_final_kernel = pl.pallas_call(
    _final_body,
    out_shape=jax.ShapeDtypeStruct((1, 1), jnp.float32),
)


def kernel(pred, target):
    p = pred.reshape(DROWS, 128)
    t = target.reshape(DROWS, 128).astype(jnp.int32)
    hists = _hist_kernel(p, t)
    return _final_kernel(hists).reshape(())
